# baseline (speedup 1.0000x reference)
import functools

import jax
import jax.numpy as jnp
from jax import lax
from jax.experimental import pallas as pl
from jax.experimental.pallas import tpu as pltpu
from jax.experimental.pallas import tpu_sc as plsc

NC, NS, L = 2, 16, 16
NW = NC * NS
BSZ, SEQ, NA, D = 2, 2048, 8, 1024
ROWS = BSZ * SEQ
RPW = ROWS // NW
CH = 16
NBUF = 3
NCHUNK = RPW // CH
WPB = NW // BSZ


def _body(ao, res, idxb, out, idx_v, idx_vv, *bufs_and_sems):
    gbuf = bufs_and_sems[0:NBUF]
    rbuf = bufs_and_sems[NBUF:2 * NBUF]
    gsem = bufs_and_sems[2 * NBUF:3 * NBUF]
    rsem = bufs_and_sems[3 * NBUF:4 * NBUF]
    osem = bufs_and_sems[4 * NBUF:5 * NBUF]

    c = lax.axis_index("c")
    s = lax.axis_index("s")
    wid = s * NC + c
    b = wid // WPB
    base = wid * RPW

    pltpu.sync_copy(idxb.at[b], idx_vv)
    iv = idx_vv[...]

    def build_idx(ch):
        for j in range(CH // L):
            o = ch * CH + j * L
            idx_v[pl.ds(o, L)] = (
                base + o + lax.iota(jnp.int32, L)) * NA + iv

    def issue_in(ch):
        nb = ch % NBUF
        g = pltpu.async_copy(
            ao.at[idx_v.at[pl.ds(ch * CH, CH)]], gbuf[nb], gsem[nb])
        r = pltpu.async_copy(
            res.at[pl.ds(base + ch * CH, CH)], rbuf[nb], rsem[nb])
        return g, r

    pend_in = {}
    pend_out = {}
    for ch in range(NBUF - 1):
        build_idx(ch)
        pend_in[ch] = issue_in(ch)
    for ch in range(NBUF - 1, NCHUNK):
        build_idx(ch)

    for ch in range(NCHUNK):
        nb = ch % NBUF
        ahead = ch + NBUF - 1
        if ahead < NCHUNK:
            if ahead - NBUF in pend_out:
                pend_out.pop(ahead - NBUF).wait()
            pend_in[ahead] = issue_in(ahead)
        g, r = pend_in.pop(ch)
        g.wait()
        r.wait()

        def row_add(i, carry, _g=gbuf[nb], _r=rbuf[nb]):
            for k in range(2):
                for j in range(D // L):
                    sl = pl.ds(j * L, L)
                    plsc.addupdate(_r.at[2 * i + k, sl], _g[2 * i + k, sl])
            return carry

        lax.fori_loop(0, CH // 2, row_add, 0)
        pend_out[ch] = pltpu.async_copy(
            rbuf[nb], out.at[pl.ds(base + ch * CH, CH)], osem[nb])
    for cp in pend_out.values():
        cp.wait()


_sc_call = functools.partial(
    pl.kernel,
    out_type=jax.ShapeDtypeStruct((ROWS, D), jnp.float32),
    mesh=plsc.VectorSubcoreMesh(core_axis_name="c", subcore_axis_name="s"),
    scratch_types=(
        [pltpu.VMEM((RPW,), jnp.int32),
         pltpu.VMEM((L,), jnp.int32)]
        + [pltpu.VMEM((CH, D), jnp.float32)] * (2 * NBUF)
        + [pltpu.SemaphoreType.DMA] * (3 * NBUF)
    ),
)(_body)


def kernel(inputs, adapter_outputs, adapter_outputs_copy, residual, adapter_indices):
    ao = adapter_outputs.reshape(ROWS * NA, D)
    res = residual.reshape(ROWS, D)
    idxb = jnp.broadcast_to(
        adapter_indices.astype(jnp.int32)[:, None], (BSZ, L))
    out = _sc_call(ao, res, idxb)
    return out.reshape(BSZ, SEQ, D)


# speedup vs baseline: 1.1093x; 1.1093x over previous
import functools

import jax
import jax.numpy as jnp
from jax import lax
from jax.experimental import pallas as pl
from jax.experimental.pallas import tpu as pltpu
from jax.experimental.pallas import tpu_sc as plsc

NC, NS, L = 2, 16, 16
NW = NC * NS
BSZ, SEQ, NA, D = 2, 2048, 8, 1024
ROWS = BSZ * SEQ
RPW = ROWS // NW
CH = 16
NBUF = 3
NCHUNK = RPW // CH
WPB = NW // BSZ


def _body(ao, res, idxb, out, idx_v, idx_vv, *bufs_and_sems):
    gbuf = bufs_and_sems[0:NBUF]
    rbuf = bufs_and_sems[NBUF:2 * NBUF]
    gsem = bufs_and_sems[2 * NBUF:3 * NBUF]
    rsem = bufs_and_sems[3 * NBUF:4 * NBUF]
    osem = bufs_and_sems[4 * NBUF:5 * NBUF]

    c = lax.axis_index("c")
    s = lax.axis_index("s")
    wid = s * NC + c
    b = wid // WPB
    base = wid * RPW

    pltpu.sync_copy(idxb.at[b], idx_vv)
    iv = idx_vv[...]

    def build_idx(ch):
        for j in range(CH // L):
            o = ch * CH + j * L
            idx_v[pl.ds(o, L)] = (
                base + o + lax.iota(jnp.int32, L)) * NA + iv

    def issue_in(ch):
        nb = ch % NBUF
        g = pltpu.async_copy(
            ao.at[idx_v.at[pl.ds(ch * CH, CH)]], gbuf[nb], gsem[nb])
        r = pltpu.async_copy(
            res.at[pl.ds(base + ch * CH, CH)], rbuf[nb], rsem[nb])
        return g, r

    pend_in = {}
    pend_out = {}
    for ch in range(NBUF - 1):
        build_idx(ch)
        pend_in[ch] = issue_in(ch)
    for ch in range(NBUF - 1, NCHUNK):
        build_idx(ch)

    for ch in range(NCHUNK):
        nb = ch % NBUF
        ahead = ch + NBUF - 1
        if ahead < NCHUNK:
            if ahead - NBUF in pend_out:
                pend_out.pop(ahead - NBUF).wait()
            pend_in[ahead] = issue_in(ahead)
        g, r = pend_in.pop(ch)
        g.wait()
        r.wait()

        def row_add(i, carry, _g=gbuf[nb], _r=rbuf[nb]):
            for j in range(D // L):
                sl = pl.ds(j * L, L)
                plsc.addupdate(_r.at[i, sl], _g[i, sl])
            return carry

        lax.fori_loop(0, CH, row_add, 0)
        pend_out[ch] = pltpu.async_copy(
            rbuf[nb], out.at[pl.ds(base + ch * CH, CH)], osem[nb])
    for cp in pend_out.values():
        cp.wait()


_sc_call = functools.partial(
    pl.kernel,
    out_type=jax.ShapeDtypeStruct((ROWS, D), jnp.float32),
    mesh=plsc.VectorSubcoreMesh(core_axis_name="c", subcore_axis_name="s"),
    scratch_types=(
        [pltpu.VMEM((RPW,), jnp.int32),
         pltpu.VMEM((L,), jnp.int32)]
        + [pltpu.VMEM((CH, D), jnp.float32)] * (2 * NBUF)
        + [pltpu.SemaphoreType.DMA] * (3 * NBUF)
    ),
)(_body)


def kernel(inputs, adapter_outputs, adapter_outputs_copy, residual, adapter_indices):
    ao = adapter_outputs.reshape(ROWS * NA, D)
    res = residual.reshape(ROWS, D)
    idxb = jnp.broadcast_to(
        adapter_indices.astype(jnp.int32)[:, None], (BSZ, L))
    out = _sc_call(ao, res, idxb)
    return out.reshape(BSZ, SEQ, D)
